# baseline (speedup 1.0000x reference)
import functools

import jax
import jax.numpy as jnp
from jax import lax
from jax.experimental import pallas as pl
from jax.experimental.pallas import tpu as pltpu
from jax.experimental.pallas import tpu_sc as plsc

D_MODEL = 1024
BATCH = 4
SEQ = 2048
B = BATCH * SEQ

_NC = 2
_NS = 16
_NW = _NC * _NS
_BPW = B // _NW
_CHUNKS = (96, 96, 64)
_BUF = max(_CHUNKS)

_mesh = plsc.VectorSubcoreMesh(core_axis_name="c", subcore_axis_name="s")


@functools.partial(
    pl.kernel,
    mesh=_mesh,
    out_type=jax.ShapeDtypeStruct((B, D_MODEL), jnp.float32),
    scratch_types=[
        pltpu.VMEM((_BPW,), jnp.int32),
        pltpu.VMEM((_BUF, D_MODEL), jnp.float32),
        pltpu.SemaphoreType.DMA,
        pltpu.SemaphoreType.DMA,
    ],
)
def _embed_sc(ids_hbm, table_hbm, out_hbm, idx_v, rows_v, gsem, osem):
    wid = lax.axis_index("s") * _NC + lax.axis_index("c")
    base = wid * _BPW
    pltpu.sync_copy(ids_hbm.at[pl.ds(base, _BPW)], idx_v)

    off = 0
    last_put = None
    for c in _CHUNKS:
        if last_put is not None:
            last_put.wait()
        pltpu.async_copy(
            table_hbm.at[idx_v.at[pl.ds(off, c)]], rows_v.at[pl.ds(0, c)], gsem
        ).wait()
        last_put = pltpu.async_copy(
            rows_v.at[pl.ds(0, c)], out_hbm.at[pl.ds(base + off, c)], osem
        )
        off += c
    last_put.wait()


def kernel(text_ids, table):
    ids_flat = text_ids.reshape(-1).astype(jnp.int32)
    out = _embed_sc(ids_flat, table)
    return out.reshape(BATCH, SEQ, D_MODEL)
